# Initial kernel scaffold; baseline (speedup 1.0000x reference)
import jax
import jax.numpy as jnp
from jax.experimental import pallas as pl


def kernel(sentence_tokens, embedding_table, linear_w, linear_b):
    raise NotImplementedError("write your pallas kernel here")


# speedup vs baseline: 20.5637x; 20.5637x over previous
import functools

import jax
import jax.numpy as jnp
from jax import lax
from jax.experimental import pallas as pl
from jax.experimental.pallas import tpu as pltpu
from jax.experimental.pallas import tpu_sc as plsc

V = 100000
D = 64
S = 200
B = 4096

NC = 2
NS = 16
SC_CHUNK = 40
N_CHUNKS = S // SC_CHUNK
BC = B // NS
NG = BC // 16


def _proj_body(w_ref, t_ref, o_ref):
    o_ref[...] = lax.dot_general(
        w_ref[...], t_ref[...], (((1,), (1,)), ((), ())),
        preferred_element_type=jnp.float32,
        precision=lax.Precision.HIGHEST)


def _project_table(linear_w, embedding_table):
    return pl.pallas_call(
        _proj_body,
        out_shape=jax.ShapeDtypeStruct((2, V), jnp.float32),
    )(linear_w, embedding_table)


def _sc_body(tok_hbm, proj_hbm, out_hbm, proj_v, buf0, buf1, acc_v, sem0, sem1):
    cls = lax.axis_index("c")
    sid = lax.axis_index("s")
    base = sid * BC
    bufs = (buf0, buf1)
    sems = (sem0, sem1)

    cps = [None, None]
    cps[0] = pltpu.async_copy(
        tok_hbm.at[pl.ds(0, SC_CHUNK), pl.ds(base, BC)], bufs[0], sems[0])
    pltpu.sync_copy(proj_hbm.at[cls], proj_v)

    for k in range(N_CHUNKS):
        if k + 1 < N_CHUNKS:
            nb = (k + 1) % 2
            cps[nb] = pltpu.async_copy(
                tok_hbm.at[pl.ds((k + 1) * SC_CHUNK, SC_CHUNK), pl.ds(base, BC)],
                bufs[nb], sems[nb])
        cps[k % 2].wait()
        buf = bufs[k % 2]
        for g in range(NG):
            if k == 0:
                acc = jnp.zeros((16,), jnp.float32)
            else:
                acc = acc_v[pl.ds(g * 16, 16)]

            def sbody(s, a, _buf=buf, _g=g):
                tok = _buf[s, pl.ds(_g * 16, 16)]
                return a + plsc.load_gather(proj_v, [tok])

            acc = lax.fori_loop(0, SC_CHUNK, sbody, acc, unroll=5)
            acc_v[pl.ds(g * 16, 16)] = acc

    pltpu.sync_copy(acc_v, out_hbm.at[cls, pl.ds(base, BC)])


def _sc_scores(tokens, proj):
    mesh = plsc.VectorSubcoreMesh(core_axis_name="c", subcore_axis_name="s")
    f = functools.partial(
        pl.kernel,
        out_type=jax.ShapeDtypeStruct((2, B), jnp.float32),
        mesh=mesh,
        scratch_types=[
            pltpu.VMEM((V,), jnp.float32),
            pltpu.VMEM((SC_CHUNK, BC), jnp.int32),
            pltpu.VMEM((SC_CHUNK, BC), jnp.int32),
            pltpu.VMEM((BC,), jnp.float32),
            pltpu.SemaphoreType.DMA,
            pltpu.SemaphoreType.DMA,
        ],
        compiler_params=pltpu.CompilerParams(needs_layout_passes=False),
    )(_sc_body)
    return f(tokens, proj)


def _lsm_body(s_ref, b_ref, o_ref):
    x = s_ref[...] + b_ref[...]
    m = jnp.max(x, axis=1, keepdims=True)
    lse = m + jnp.log(jnp.sum(jnp.exp(x - m), axis=1, keepdims=True))
    o_ref[...] = x - lse


def _log_softmax(scores_t, linear_b):
    return pl.pallas_call(
        _lsm_body,
        out_shape=jax.ShapeDtypeStruct((2, B), jnp.float32),
    )(scores_t, linear_b.reshape(2, 1))


def kernel(sentence_tokens, embedding_table, linear_w, linear_b):
    tokens = sentence_tokens.astype(jnp.int32)
    proj = _project_table(linear_w, embedding_table)
    scores_t = _sc_scores(tokens, proj)
    logp_t = _log_softmax(scores_t, linear_b)
    return logp_t.T


# speedup vs baseline: 21.2393x; 1.0329x over previous
import functools

import jax
import jax.numpy as jnp
from jax import lax
from jax.experimental import pallas as pl
from jax.experimental.pallas import tpu as pltpu
from jax.experimental.pallas import tpu_sc as plsc

V = 100000
D = 64
S = 200
B = 4096

NC = 2
NS = 16
SC_CHUNK = 40
N_CHUNKS = S // SC_CHUNK
BC = B // NS
NG = BC // 16


VB = 2048
NVB = 49
PV = NVB * VB


def _proj_body(w_ref, t_ref, o_ref):
    o_ref[...] = lax.dot_general(
        w_ref[...], t_ref[...], (((1,), (1,)), ((), ())),
        preferred_element_type=jnp.float32,
        precision=lax.Precision.HIGHEST)


def _project_table(linear_w, embedding_table):
    return pl.pallas_call(
        _proj_body,
        grid=(NVB,),
        in_specs=[
            pl.BlockSpec((2, D), lambda i: (0, 0)),
            pl.BlockSpec((VB, D), lambda i: (i, 0)),
        ],
        out_specs=pl.BlockSpec((2, VB), lambda i: (0, i)),
        out_shape=jax.ShapeDtypeStruct((2, PV), jnp.float32),
    )(linear_w, embedding_table)


def _sc_body(tok_hbm, proj_hbm, out_hbm, proj_v, buf0, buf1, acc_v,
             sem0, sem1, semp):
    cls = lax.axis_index("c")
    sid = lax.axis_index("s")
    base = sid * BC
    bufs = (buf0, buf1)
    sems = (sem0, sem1)

    cps = [None, None]
    cps[0] = pltpu.async_copy(
        tok_hbm.at[pl.ds(0, SC_CHUNK), pl.ds(base, BC)], bufs[0], sems[0])
    pltpu.async_copy(proj_hbm.at[cls], proj_v, semp).wait()

    for k in range(N_CHUNKS):
        if k + 1 < N_CHUNKS:
            nb = (k + 1) % 2
            cps[nb] = pltpu.async_copy(
                tok_hbm.at[pl.ds((k + 1) * SC_CHUNK, SC_CHUNK), pl.ds(base, BC)],
                bufs[nb], sems[nb])
        cps[k % 2].wait()
        buf = bufs[k % 2]
        for g in range(NG):
            if k == 0:
                acc = jnp.zeros((16,), jnp.float32)
            else:
                acc = acc_v[pl.ds(g * 16, 16)]

            def sbody(s, a, _buf=buf, _g=g):
                tok = _buf[s, pl.ds(_g * 16, 16)]
                return a + plsc.load_gather(proj_v, [tok])

            acc = lax.fori_loop(0, SC_CHUNK, sbody, acc, unroll=5)
            acc_v[pl.ds(g * 16, 16)] = acc

    pltpu.sync_copy(acc_v, out_hbm.at[cls, pl.ds(base, BC)])


def _sc_scores(tokens, proj):
    mesh = plsc.VectorSubcoreMesh(core_axis_name="c", subcore_axis_name="s")
    f = functools.partial(
        pl.kernel,
        out_type=jax.ShapeDtypeStruct((2, B), jnp.float32),
        mesh=mesh,
        scratch_types=[
            pltpu.VMEM((PV,), jnp.float32),
            pltpu.VMEM((SC_CHUNK, BC), jnp.int32),
            pltpu.VMEM((SC_CHUNK, BC), jnp.int32),
            pltpu.VMEM((BC,), jnp.float32),
            pltpu.SemaphoreType.DMA,
            pltpu.SemaphoreType.DMA,
            pltpu.SemaphoreType.DMA,
        ],
        compiler_params=pltpu.CompilerParams(needs_layout_passes=False),
    )(_sc_body)
    return f(tokens, proj)


def _lsm_body(s_ref, b_ref, o_ref):
    x = s_ref[...] + b_ref[...]
    m = jnp.max(x, axis=1, keepdims=True)
    lse = m + jnp.log(jnp.sum(jnp.exp(x - m), axis=1, keepdims=True))
    o_ref[...] = x - lse


def _log_softmax(scores_t, linear_b):
    return pl.pallas_call(
        _lsm_body,
        out_shape=jax.ShapeDtypeStruct((2, B), jnp.float32),
    )(scores_t, linear_b.reshape(2, 1))


def kernel(sentence_tokens, embedding_table, linear_w, linear_b):
    tokens = sentence_tokens.astype(jnp.int32)
    proj = _project_table(linear_w, embedding_table)
    scores_t = _sc_scores(tokens, proj)
    logp_t = _log_softmax(scores_t, linear_b)
    return logp_t.T


# speedup vs baseline: 27.7102x; 1.3047x over previous
import functools

import jax
import jax.numpy as jnp
from jax import lax
from jax.experimental import pallas as pl
from jax.experimental.pallas import tpu as pltpu
from jax.experimental.pallas import tpu_sc as plsc

V = 100000
D = 64
S = 200
B = 4096

NC = 2
NS = 16
SC_CHUNK = 40
N_CHUNKS = S // SC_CHUNK
BC = B // NS
NG = BC // 16


VB = 4096
NVB = 25
PV = NVB * VB


def _proj_body(w_ref, t_ref, o_ref):
    o_ref[...] = lax.dot_general(
        w_ref[...], t_ref[...], (((1,), (1,)), ((), ())),
        preferred_element_type=jnp.float32)


def _project_table(linear_w, embedding_table):
    return pl.pallas_call(
        _proj_body,
        grid=(NVB,),
        in_specs=[
            pl.BlockSpec((2, D), lambda i: (0, 0)),
            pl.BlockSpec((VB, D), lambda i: (i, 0)),
        ],
        out_specs=pl.BlockSpec((2, VB), lambda i: (0, i)),
        out_shape=jax.ShapeDtypeStruct((2, PV), jnp.float32),
    )(linear_w, embedding_table)


def _sc_body(tok_hbm, proj_hbm, out_hbm, proj_v, proj_sh, buf0, buf1, acc_v,
             sem0, sem1, semp):
    cls = lax.axis_index("c")
    sid = lax.axis_index("s")
    base = sid * BC
    bufs = (buf0, buf1)
    sems = (sem0, sem1)

    cps = [None, None]
    cps[0] = pltpu.async_copy(
        tok_hbm.at[pl.ds(0, SC_CHUNK), pl.ds(base, BC)], bufs[0], sems[0])

    @pl.when(sid == 0)
    def _():
        pltpu.async_copy(proj_hbm.at[cls], proj_sh, semp).wait()

    plsc.subcore_barrier()
    pltpu.sync_copy(proj_sh, proj_v)

    for k in range(N_CHUNKS):
        if k + 1 < N_CHUNKS:
            nb = (k + 1) % 2
            cps[nb] = pltpu.async_copy(
                tok_hbm.at[pl.ds((k + 1) * SC_CHUNK, SC_CHUNK), pl.ds(base, BC)],
                bufs[nb], sems[nb])
        cps[k % 2].wait()
        buf = bufs[k % 2]
        for g in range(NG):
            if k == 0:
                acc = jnp.zeros((16,), jnp.float32)
            else:
                acc = acc_v[pl.ds(g * 16, 16)]

            def sbody(s, a, _buf=buf, _g=g):
                tok = _buf[s, pl.ds(_g * 16, 16)]
                return a + plsc.load_gather(proj_v, [tok])

            acc = lax.fori_loop(0, SC_CHUNK, sbody, acc, unroll=5)
            acc_v[pl.ds(g * 16, 16)] = acc

    pltpu.sync_copy(acc_v, out_hbm.at[cls, pl.ds(base, BC)])


def _sc_scores(tokens, proj):
    mesh = plsc.VectorSubcoreMesh(core_axis_name="c", subcore_axis_name="s")
    f = functools.partial(
        pl.kernel,
        out_type=jax.ShapeDtypeStruct((2, B), jnp.float32),
        mesh=mesh,
        scratch_types=[
            pltpu.VMEM((PV,), jnp.float32),
            pltpu.VMEM_SHARED((PV,), jnp.float32),
            pltpu.VMEM((SC_CHUNK, BC), jnp.int32),
            pltpu.VMEM((SC_CHUNK, BC), jnp.int32),
            pltpu.VMEM((BC,), jnp.float32),
            pltpu.SemaphoreType.DMA,
            pltpu.SemaphoreType.DMA,
            pltpu.SemaphoreType.DMA,
        ],
        compiler_params=pltpu.CompilerParams(needs_layout_passes=False),
    )(_sc_body)
    return f(tokens, proj)


def _lsm_body(s_ref, b_ref, o_ref):
    x = s_ref[...] + b_ref[...]
    m = jnp.max(x, axis=1, keepdims=True)
    lse = m + jnp.log(jnp.sum(jnp.exp(x - m), axis=1, keepdims=True))
    o_ref[...] = x - lse


def _log_softmax(scores_t, linear_b):
    return pl.pallas_call(
        _lsm_body,
        out_shape=jax.ShapeDtypeStruct((2, B), jnp.float32),
    )(scores_t, linear_b.reshape(2, 1))


def kernel(sentence_tokens, embedding_table, linear_w, linear_b):
    tokens = sentence_tokens.astype(jnp.int32)
    proj = _project_table(linear_w, embedding_table)
    scores_t = _sc_scores(tokens, proj)
    logp_t = _log_softmax(scores_t, linear_b)
    return logp_t.T


# speedup vs baseline: 27.9408x; 1.0083x over previous
import functools

import jax
import jax.numpy as jnp
from jax import lax
from jax.experimental import pallas as pl
from jax.experimental.pallas import tpu as pltpu
from jax.experimental.pallas import tpu_sc as plsc

V = 100000
D = 64
S = 200
B = 4096

NC = 2
NS = 16
SC_CHUNK = 40
N_CHUNKS = S // SC_CHUNK
BC = B // NS
NG = BC // 16


VB = 8192
NVB = 13
PV = NVB * VB


def _proj_body(w_ref, t_ref, o_ref):
    o_ref[...] = lax.dot_general(
        w_ref[...], t_ref[...], (((1,), (1,)), ((), ())),
        preferred_element_type=jnp.float32)


def _project_table(linear_w, embedding_table):
    return pl.pallas_call(
        _proj_body,
        grid=(NVB,),
        in_specs=[
            pl.BlockSpec((2, D), lambda i: (0, 0)),
            pl.BlockSpec((VB, D), lambda i: (i, 0)),
        ],
        out_specs=pl.BlockSpec((2, VB), lambda i: (0, i)),
        out_shape=jax.ShapeDtypeStruct((2, PV), jnp.float32),
    )(linear_w, embedding_table)


def _sc_body(tok_hbm, proj_hbm, out_hbm, proj_v, buf0, buf1, acc_v,
             sem0, sem1, semp):
    cls = lax.axis_index("c")
    sid = lax.axis_index("s")
    base = sid * BC
    bufs = (buf0, buf1)
    sems = (sem0, sem1)

    cps = [None, None]
    cps[0] = pltpu.async_copy(
        tok_hbm.at[pl.ds(0, SC_CHUNK), pl.ds(base, BC)], bufs[0], sems[0])
    pltpu.async_copy(proj_hbm.at[cls], proj_v, semp).wait()

    for k in range(N_CHUNKS):
        if k + 1 < N_CHUNKS:
            nb = (k + 1) % 2
            cps[nb] = pltpu.async_copy(
                tok_hbm.at[pl.ds((k + 1) * SC_CHUNK, SC_CHUNK), pl.ds(base, BC)],
                bufs[nb], sems[nb])
        cps[k % 2].wait()
        buf = bufs[k % 2]
        for g in range(NG):
            if k == 0:
                acc = jnp.zeros((16,), jnp.float32)
            else:
                acc = acc_v[pl.ds(g * 16, 16)]

            def sbody(s, a, _buf=buf, _g=g):
                tok = _buf[s, pl.ds(_g * 16, 16)]
                return a + plsc.load_gather(proj_v, [tok])

            acc = lax.fori_loop(0, SC_CHUNK, sbody, acc, unroll=5)
            acc_v[pl.ds(g * 16, 16)] = acc

    pltpu.sync_copy(acc_v, out_hbm.at[cls, pl.ds(base, BC)])


def _sc_scores(tokens, proj):
    mesh = plsc.VectorSubcoreMesh(core_axis_name="c", subcore_axis_name="s")
    f = functools.partial(
        pl.kernel,
        out_type=jax.ShapeDtypeStruct((2, B), jnp.float32),
        mesh=mesh,
        scratch_types=[
            pltpu.VMEM((PV,), jnp.float32),
            pltpu.VMEM((SC_CHUNK, BC), jnp.int32),
            pltpu.VMEM((SC_CHUNK, BC), jnp.int32),
            pltpu.VMEM((BC,), jnp.float32),
            pltpu.SemaphoreType.DMA,
            pltpu.SemaphoreType.DMA,
            pltpu.SemaphoreType.DMA,
        ],
        compiler_params=pltpu.CompilerParams(needs_layout_passes=False),
    )(_sc_body)
    return f(tokens, proj)


def _lsm_body(s_ref, b_ref, o_ref):
    x = s_ref[...] + b_ref[...]
    m = jnp.max(x, axis=1, keepdims=True)
    lse = m + jnp.log(jnp.sum(jnp.exp(x - m), axis=1, keepdims=True))
    o_ref[...] = x - lse


def _log_softmax(scores_t, linear_b):
    return pl.pallas_call(
        _lsm_body,
        out_shape=jax.ShapeDtypeStruct((2, B), jnp.float32),
    )(scores_t, linear_b.reshape(2, 1))


def kernel(sentence_tokens, embedding_table, linear_w, linear_b):
    tokens = sentence_tokens.astype(jnp.int32)
    proj = _project_table(linear_w, embedding_table)
    scores_t = _sc_scores(tokens, proj)
    logp_t = _log_softmax(scores_t, linear_b)
    return logp_t.T


# speedup vs baseline: 28.7154x; 1.0277x over previous
import functools

import jax
import jax.numpy as jnp
from jax import lax
from jax.experimental import pallas as pl
from jax.experimental.pallas import tpu as pltpu
from jax.experimental.pallas import tpu_sc as plsc

V = 100000
D = 64
S = 200
B = 4096

NC = 2
NS = 16
SC_CHUNK = 40
N_CHUNKS = S // SC_CHUNK
BC = B // NS
NG = BC // 16


VB = 4096
NVB = 25
PV = NVB * VB
V_LAST = V - (NVB - 1) * VB
NBUF = 4


def _proj_body(w_ref, t_hbm, o_ref, *scratch):
    tbufs = scratch[:NBUF]
    sems = scratch[NBUF:]

    def start(j):
        rows = V_LAST if j == NVB - 1 else VB
        pltpu.make_async_copy(
            t_hbm.at[pl.ds(j * VB, rows)], tbufs[j % NBUF].at[pl.ds(0, rows)],
            sems[j % NBUF]).start()

    def wait(j):
        rows = V_LAST if j == NVB - 1 else VB
        pltpu.make_async_copy(
            t_hbm.at[pl.ds(j * VB, rows)], tbufs[j % NBUF].at[pl.ds(0, rows)],
            sems[j % NBUF]).wait()

    w = w_ref[...]
    for j in range(NBUF):
        start(j)
    for j in range(NVB):
        wait(j)
        o_ref[:, pl.ds(j * VB, VB)] = lax.dot_general(
            w, tbufs[j % NBUF][...], (((1,), (1,)), ((), ())),
            preferred_element_type=jnp.float32)
        if j + NBUF < NVB:
            start(j + NBUF)


def _project_table(linear_w, embedding_table):
    return pl.pallas_call(
        _proj_body,
        in_specs=[
            pl.BlockSpec(memory_space=pltpu.VMEM),
            pl.BlockSpec(memory_space=pl.ANY),
        ],
        out_specs=pl.BlockSpec(memory_space=pltpu.VMEM),
        out_shape=jax.ShapeDtypeStruct((2, PV), jnp.float32),
        scratch_shapes=(
            [pltpu.VMEM((VB, D), jnp.float32) for _ in range(NBUF)]
            + [pltpu.SemaphoreType.DMA for _ in range(NBUF)]
        ),
    )(linear_w, embedding_table)


def _sc_body(tok_hbm, proj_hbm, out_hbm, proj_v, buf0, buf1, acc_v,
             sem0, sem1, semp):
    cls = lax.axis_index("c")
    sid = lax.axis_index("s")
    base = sid * BC
    bufs = (buf0, buf1)
    sems = (sem0, sem1)

    cps = [None, None]
    cps[0] = pltpu.async_copy(
        tok_hbm.at[pl.ds(0, SC_CHUNK), pl.ds(base, BC)], bufs[0], sems[0])
    pltpu.async_copy(proj_hbm.at[cls], proj_v, semp).wait()

    for k in range(N_CHUNKS):
        if k + 1 < N_CHUNKS:
            nb = (k + 1) % 2
            cps[nb] = pltpu.async_copy(
                tok_hbm.at[pl.ds((k + 1) * SC_CHUNK, SC_CHUNK), pl.ds(base, BC)],
                bufs[nb], sems[nb])
        cps[k % 2].wait()
        buf = bufs[k % 2]
        for g in range(NG):
            if k == 0:
                acc = jnp.zeros((16,), jnp.float32)
            else:
                acc = acc_v[pl.ds(g * 16, 16)]

            def sbody(s, a, _buf=buf, _g=g):
                tok = _buf[s, pl.ds(_g * 16, 16)]
                return a + plsc.load_gather(proj_v, [tok])

            acc = lax.fori_loop(0, SC_CHUNK, sbody, acc, unroll=5)
            acc_v[pl.ds(g * 16, 16)] = acc

    pltpu.sync_copy(acc_v, out_hbm.at[cls, pl.ds(base, BC)])


def _sc_scores(tokens, proj):
    mesh = plsc.VectorSubcoreMesh(core_axis_name="c", subcore_axis_name="s")
    f = functools.partial(
        pl.kernel,
        out_type=jax.ShapeDtypeStruct((2, B), jnp.float32),
        mesh=mesh,
        scratch_types=[
            pltpu.VMEM((PV,), jnp.float32),
            pltpu.VMEM((SC_CHUNK, BC), jnp.int32),
            pltpu.VMEM((SC_CHUNK, BC), jnp.int32),
            pltpu.VMEM((BC,), jnp.float32),
            pltpu.SemaphoreType.DMA,
            pltpu.SemaphoreType.DMA,
            pltpu.SemaphoreType.DMA,
        ],
        compiler_params=pltpu.CompilerParams(needs_layout_passes=False),
    )(_sc_body)
    return f(tokens, proj)


def _lsm_body(s_ref, b_ref, o_ref):
    x = s_ref[...] + b_ref[...]
    m = jnp.max(x, axis=1, keepdims=True)
    lse = m + jnp.log(jnp.sum(jnp.exp(x - m), axis=1, keepdims=True))
    o_ref[...] = x - lse


def _log_softmax(scores_t, linear_b):
    return pl.pallas_call(
        _lsm_body,
        out_shape=jax.ShapeDtypeStruct((2, B), jnp.float32),
    )(scores_t, linear_b.reshape(2, 1))


def kernel(sentence_tokens, embedding_table, linear_w, linear_b):
    tokens = sentence_tokens.astype(jnp.int32)
    proj = _project_table(linear_w, embedding_table)
    scores_t = _sc_scores(tokens, proj)
    logp_t = _log_softmax(scores_t, linear_b)
    return logp_t.T


# speedup vs baseline: 30.5978x; 1.0656x over previous
import functools

import jax
import jax.numpy as jnp
from jax import lax
from jax.experimental import pallas as pl
from jax.experimental.pallas import tpu as pltpu
from jax.experimental.pallas import tpu_sc as plsc

V = 100000
D = 64
S = 200
B = 4096

NC = 2
NS = 16
SC_CHUNK = 40
N_CHUNKS = S // SC_CHUNK
BC = B // NS
NG = BC // 16


VB = 4096
NVB = 25
PV = NVB * VB
V_LAST = V - (NVB - 1) * VB
NBUF = 4


def _proj_body(w_ref, t_hbm, o_ref, *scratch):
    tbufs = scratch[:NBUF]
    sems = scratch[NBUF:]

    def start(j):
        rows = V_LAST if j == NVB - 1 else VB
        pltpu.make_async_copy(
            t_hbm.at[pl.ds(j * VB, rows)], tbufs[j % NBUF].at[pl.ds(0, rows)],
            sems[j % NBUF]).start()

    def wait(j):
        rows = V_LAST if j == NVB - 1 else VB
        pltpu.make_async_copy(
            t_hbm.at[pl.ds(j * VB, rows)], tbufs[j % NBUF].at[pl.ds(0, rows)],
            sems[j % NBUF]).wait()

    w = w_ref[...]
    for j in range(NBUF):
        start(j)
    for j in range(NVB):
        wait(j)
        o_ref[:, pl.ds(j * VB, VB)] = lax.dot_general(
            w, tbufs[j % NBUF][...], (((1,), (1,)), ((), ())),
            preferred_element_type=jnp.float32)
        if j + NBUF < NVB:
            start(j + NBUF)


def _project_table(linear_w, embedding_table):
    return pl.pallas_call(
        _proj_body,
        in_specs=[
            pl.BlockSpec(memory_space=pltpu.VMEM),
            pl.BlockSpec(memory_space=pl.ANY),
        ],
        out_specs=pl.BlockSpec(memory_space=pltpu.VMEM),
        out_shape=jax.ShapeDtypeStruct((2, PV), jnp.float32),
        scratch_shapes=(
            [pltpu.VMEM((VB, D), jnp.float32) for _ in range(NBUF)]
            + [pltpu.SemaphoreType.DMA for _ in range(NBUF)]
        ),
    )(linear_w, embedding_table)


def _sc_body(tok_hbm, proj_hbm, out_hbm, proj_v, proj_sh, buf0, buf1, acc_v,
             sem0, sem1, semp):
    cls = lax.axis_index("c")
    sid = lax.axis_index("s")
    base = sid * BC
    bufs = (buf0, buf1)
    sems = (sem0, sem1)

    cps = [None, None]
    cps[0] = pltpu.async_copy(
        tok_hbm.at[pl.ds(0, SC_CHUNK), pl.ds(base, BC)], bufs[0], sems[0])

    @pl.when(sid == 0)
    def _():
        pltpu.async_copy(proj_hbm.at[cls], proj_sh, semp).wait()

    plsc.subcore_barrier()
    pltpu.sync_copy(proj_sh, proj_v)

    for k in range(N_CHUNKS):
        if k + 1 < N_CHUNKS:
            nb = (k + 1) % 2
            cps[nb] = pltpu.async_copy(
                tok_hbm.at[pl.ds((k + 1) * SC_CHUNK, SC_CHUNK), pl.ds(base, BC)],
                bufs[nb], sems[nb])
        cps[k % 2].wait()
        buf = bufs[k % 2]
        for g in range(NG):
            if k == 0:
                acc = jnp.zeros((16,), jnp.float32)
            else:
                acc = acc_v[pl.ds(g * 16, 16)]

            def sbody(s, a, _buf=buf, _g=g):
                tok = _buf[s, pl.ds(_g * 16, 16)]
                return a + plsc.load_gather(proj_v, [tok])

            acc = lax.fori_loop(0, SC_CHUNK, sbody, acc, unroll=5)
            acc_v[pl.ds(g * 16, 16)] = acc

    pltpu.sync_copy(acc_v, out_hbm.at[cls, pl.ds(base, BC)])


def _sc_scores(tokens, proj):
    mesh = plsc.VectorSubcoreMesh(core_axis_name="c", subcore_axis_name="s")
    f = functools.partial(
        pl.kernel,
        out_type=jax.ShapeDtypeStruct((2, B), jnp.float32),
        mesh=mesh,
        scratch_types=[
            pltpu.VMEM((PV,), jnp.float32),
            pltpu.VMEM_SHARED((PV,), jnp.float32),
            pltpu.VMEM((SC_CHUNK, BC), jnp.int32),
            pltpu.VMEM((SC_CHUNK, BC), jnp.int32),
            pltpu.VMEM((BC,), jnp.float32),
            pltpu.SemaphoreType.DMA,
            pltpu.SemaphoreType.DMA,
            pltpu.SemaphoreType.DMA,
        ],
        compiler_params=pltpu.CompilerParams(needs_layout_passes=False),
    )(_sc_body)
    return f(tokens, proj)


def _lsm_body(s_ref, b_ref, o_ref):
    x = s_ref[...] + b_ref[...]
    m = jnp.max(x, axis=1, keepdims=True)
    lse = m + jnp.log(jnp.sum(jnp.exp(x - m), axis=1, keepdims=True))
    o_ref[...] = x - lse


def _log_softmax(scores_t, linear_b):
    return pl.pallas_call(
        _lsm_body,
        out_shape=jax.ShapeDtypeStruct((2, B), jnp.float32),
    )(scores_t, linear_b.reshape(2, 1))


def kernel(sentence_tokens, embedding_table, linear_w, linear_b):
    tokens = sentence_tokens.astype(jnp.int32)
    proj = _project_table(linear_w, embedding_table)
    scores_t = _sc_scores(tokens, proj)
    logp_t = _log_softmax(scores_t, linear_b)
    return logp_t.T


# speedup vs baseline: 34.6851x; 1.1336x over previous
import functools

import jax
import jax.numpy as jnp
from jax import lax
from jax.experimental import pallas as pl
from jax.experimental.pallas import tpu as pltpu
from jax.experimental.pallas import tpu_sc as plsc

V = 100000
D = 64
S = 200
B = 4096

NC = 2
NS = 16
SC_CHUNK = 40
N_CHUNKS = S // SC_CHUNK
BC = B // NS
BC2 = B // (NS * NC)
NG = BC2 // 16


VB = 4096
NVB = 25
PV = NVB * VB
V_LAST = V - (NVB - 1) * VB
NBUF = 4


def _proj_body(w_ref, t_hbm, o_ref, *scratch):
    tbufs = scratch[:NBUF]
    sems = scratch[NBUF:]

    def start(j):
        rows = V_LAST if j == NVB - 1 else VB
        pltpu.make_async_copy(
            t_hbm.at[pl.ds(j * VB, rows)], tbufs[j % NBUF].at[pl.ds(0, rows)],
            sems[j % NBUF]).start()

    def wait(j):
        rows = V_LAST if j == NVB - 1 else VB
        pltpu.make_async_copy(
            t_hbm.at[pl.ds(j * VB, rows)], tbufs[j % NBUF].at[pl.ds(0, rows)],
            sems[j % NBUF]).wait()

    w = w_ref[...]
    for j in range(NBUF):
        start(j)
    for j in range(NVB):
        wait(j)
        r = lax.dot_general(
            w, tbufs[j % NBUF][...], (((1,), (1,)), ((), ())),
            preferred_element_type=jnp.float32)
        bits = lax.bitcast_convert_type(r, jnp.int32) + jnp.int32(0x8000)
        lo = lax.shift_right_logical(bits[0:1], jnp.int32(16))
        hi = bits[1:2] & jnp.int32(-65536)
        o_ref[:, pl.ds(j * VB, VB)] = lo | hi
        if j + NBUF < NVB:
            start(j + NBUF)


def _project_table(linear_w, embedding_table):
    return pl.pallas_call(
        _proj_body,
        in_specs=[
            pl.BlockSpec(memory_space=pltpu.VMEM),
            pl.BlockSpec(memory_space=pl.ANY),
        ],
        out_specs=pl.BlockSpec(memory_space=pltpu.VMEM),
        out_shape=jax.ShapeDtypeStruct((1, PV), jnp.int32),
        scratch_shapes=(
            [pltpu.VMEM((VB, D), jnp.float32) for _ in range(NBUF)]
            + [pltpu.SemaphoreType.DMA for _ in range(NBUF)]
        ),
    )(linear_w, embedding_table)


def _sc_body(tok_hbm, proj_hbm, out_hbm, proj_v, proj_sh, buf0, buf1,
             acc0_v, acc1_v, sem0, sem1, semp):
    wid = lax.axis_index("s") * NC + lax.axis_index("c")
    sid = lax.axis_index("s")
    base = wid * BC2
    bufs = (buf0, buf1)
    sems = (sem0, sem1)

    cps = [None, None]
    cps[0] = pltpu.async_copy(
        tok_hbm.at[pl.ds(0, SC_CHUNK), pl.ds(base, BC2)], bufs[0], sems[0])

    @pl.when(sid == 0)
    def _():
        pltpu.async_copy(proj_hbm.at[0], proj_sh, semp).wait()

    plsc.subcore_barrier()
    pltpu.sync_copy(proj_sh, proj_v)

    himask = jnp.full((16,), -65536, jnp.int32)
    sixteen = jnp.full((16,), 16, jnp.int32)

    for k in range(N_CHUNKS):
        if k + 1 < N_CHUNKS:
            nb = (k + 1) % 2
            cps[nb] = pltpu.async_copy(
                tok_hbm.at[pl.ds((k + 1) * SC_CHUNK, SC_CHUNK),
                           pl.ds(base, BC2)],
                bufs[nb], sems[nb])
        cps[k % 2].wait()
        buf = bufs[k % 2]
        for g in range(NG):
            if k == 0:
                a0 = jnp.zeros((16,), jnp.float32)
                a1 = jnp.zeros((16,), jnp.float32)
            else:
                a0 = acc0_v[pl.ds(g * 16, 16)]
                a1 = acc1_v[pl.ds(g * 16, 16)]

            def sbody(s, a, _buf=buf, _g=g):
                tok = _buf[s, pl.ds(_g * 16, 16)]
                u = plsc.load_gather(proj_v, [tok])
                v0 = plsc.bitcast(lax.shift_left(u, sixteen), jnp.float32)
                v1 = plsc.bitcast(u & himask, jnp.float32)
                return a[0] + v0, a[1] + v1

            a0, a1 = lax.fori_loop(0, SC_CHUNK, sbody, (a0, a1), unroll=5)
            acc0_v[pl.ds(g * 16, 16)] = a0
            acc1_v[pl.ds(g * 16, 16)] = a1

    pltpu.sync_copy(acc0_v, out_hbm.at[0, pl.ds(base, BC2)])
    pltpu.sync_copy(acc1_v, out_hbm.at[1, pl.ds(base, BC2)])


def _sc_scores(tokens, proj):
    mesh = plsc.VectorSubcoreMesh(core_axis_name="c", subcore_axis_name="s")
    f = functools.partial(
        pl.kernel,
        out_type=jax.ShapeDtypeStruct((2, B), jnp.float32),
        mesh=mesh,
        scratch_types=[
            pltpu.VMEM((PV,), jnp.int32),
            pltpu.VMEM_SHARED((PV,), jnp.int32),
            pltpu.VMEM((SC_CHUNK, BC2), jnp.int32),
            pltpu.VMEM((SC_CHUNK, BC2), jnp.int32),
            pltpu.VMEM((BC2,), jnp.float32),
            pltpu.VMEM((BC2,), jnp.float32),
            pltpu.SemaphoreType.DMA,
            pltpu.SemaphoreType.DMA,
            pltpu.SemaphoreType.DMA,
        ],
        compiler_params=pltpu.CompilerParams(needs_layout_passes=False),
    )(_sc_body)
    return f(tokens, proj)


def _lsm_body(s_ref, b_ref, o_ref):
    x = s_ref[...] + b_ref[...]
    m = jnp.max(x, axis=1, keepdims=True)
    lse = m + jnp.log(jnp.sum(jnp.exp(x - m), axis=1, keepdims=True))
    o_ref[...] = x - lse


def _log_softmax(scores_t, linear_b):
    return pl.pallas_call(
        _lsm_body,
        out_shape=jax.ShapeDtypeStruct((2, B), jnp.float32),
    )(scores_t, linear_b.reshape(2, 1))


def kernel(sentence_tokens, embedding_table, linear_w, linear_b):
    tokens = sentence_tokens.astype(jnp.int32)
    proj = _project_table(linear_w, embedding_table)
    scores_t = _sc_scores(tokens, proj)
    logp_t = _log_softmax(scores_t, linear_b)
    return logp_t.T
